# baseline (speedup 1.0000x reference)
import functools

import jax
import jax.numpy as jnp
from jax import lax
from jax.experimental import pallas as pl
from jax.experimental.pallas import tpu as pltpu
from jax.experimental.pallas import tpu_sc as plsc

D = 128
W = 2 * D + 1
WP = 264
S_DIM, B_DIM, H_DIM = 2048, 32, 1024
CH = 24
NCH = WP // CH
NBUF = 4
LOOKAHEAD = 3


def _predict_kernel(hid_ref, wp_ref, wpb_ref, vp_ref, vpb_ref,
                    p_ref, c_ref, idx_ref):
    h = hid_ref[...]
    wph = lax.dot_general(h, wp_ref[...], (((1,), (1,)), ((), ())),
                          preferred_element_type=jnp.float32)
    wph = jnp.tanh(wph + wpb_ref[...])
    vp8 = jnp.broadcast_to(vp_ref[...], (8, H_DIM))
    z = lax.dot_general(wph, vp8, (((1,), (1,)), ((), ())),
                        preferred_element_type=jnp.float32)[:, :1]
    p = S_DIM * jax.nn.sigmoid(z + vpb_ref[0, 0])
    c = lax.round(p, lax.RoundingMethod.TO_NEAREST_EVEN).astype(jnp.int32)
    p_ref[...] = p
    c_ref[...] = c
    j = lax.broadcasted_iota(jnp.int32, (B_DIM, WP), 1)
    b = lax.broadcasted_iota(jnp.int32, (B_DIM, WP), 0)
    s_abs = jnp.clip(c - D + j, 0, S_DIM - 1)
    idx_ref[...] = s_abs * B_DIM + b


NB_B = 4


def _attn_kernel(p_ref, c_ref, hid_ref, enc_ref, scaled_ref, ctx_ref):
    bpid = pl.program_id(0)
    for i in range(NB_B):
        e_rows = enc_ref[i]
        h = hid_ref[i]
        p = p_ref[bpid * NB_B + i, 0]
        c = c_ref[bpid * NB_B + i, 0]
        j = lax.broadcasted_iota(jnp.int32, (1, WP), 1)
        s_abs = c - D + j
        in_range = (s_abs >= 0) & (s_abs < S_DIM)
        in_win = j < W
        live = in_range & in_win
        scores = lax.dot_general(h, e_rows, (((1,), (1,)), ((), ())),
                                 preferred_element_type=jnp.float32)
        sc = jnp.where(live, scores, 0.0)
        m = jnp.max(jnp.where(in_win, sc, -jnp.inf))
        e = jnp.where(in_win, jnp.exp(sc - m), 0.0)
        attn = e / jnp.sum(e)
        wi = s_abs.astype(jnp.float32)
        gauss = jnp.exp((wi - p) ** 2 * (-1.0 / 8192.0))
        scaled = attn * gauss
        scaled_ref[i] = scaled
        masked = jnp.where(live, scaled, 0.0)
        ctx_ref[i] = lax.dot_general(masked, e_rows, (((1,), (0,)), ((), ())),
                                     preferred_element_type=jnp.float32)


def _make_sc_gather():
    mesh = plsc.VectorSubcoreMesh(core_axis_name="c", subcore_axis_name="s")
    info = plsc.get_sparse_core_info()
    nc = info.num_cores

    @functools.partial(
        pl.kernel, mesh=mesh,
        out_type=jax.ShapeDtypeStruct((B_DIM * WP, H_DIM), jnp.float32),
        scratch_types=(
            [pltpu.VMEM((WP,), jnp.int32)]
            + [pltpu.VMEM((CH, H_DIM), jnp.float32)] * NBUF
            + [pltpu.SemaphoreType.DMA] * (2 * NBUF)
        ),
    )
    def gather_k(idx_hbm, table_hbm, out_hbm, idx_v, *scr):
        bufs = list(scr[:NBUF])
        gsem = list(scr[NBUF:2 * NBUF])
        osem = list(scr[2 * NBUF:])
        wid = lax.axis_index("s") * nc + lax.axis_index("c")
        base = wid * WP
        pltpu.sync_copy(idx_hbm.at[pl.ds(base, WP)], idx_v)
        gat_h = [None] * NBUF
        out_h = [None] * NBUF
        for ci in range(NCH + LOOKAHEAD):
            if ci < NCH:
                k = ci % NBUF
                if out_h[k] is not None:
                    out_h[k].wait()
                gat_h[k] = pltpu.async_copy(
                    table_hbm.at[idx_v.at[pl.ds(ci * CH, CH)]], bufs[k], gsem[k])
            cj = ci - LOOKAHEAD
            if 0 <= cj < NCH:
                kj = cj % NBUF
                gat_h[kj].wait()
                out_h[kj] = pltpu.async_copy(
                    bufs[kj], out_hbm.at[pl.ds(base + cj * CH, CH)], osem[kj])
        for h in out_h:
            if h is not None:
                h.wait()

    return gather_k


def kernel(t, hidden, encoder_outputs, Wp_w, Wp_b, vp_w, vp_b):
    S, B, H = encoder_outputs.shape
    p2, c2, idx2 = pl.pallas_call(
        _predict_kernel,
        out_shape=(
            jax.ShapeDtypeStruct((B, 1), jnp.float32),
            jax.ShapeDtypeStruct((B, 1), jnp.int32),
            jax.ShapeDtypeStruct((B, WP), jnp.int32),
        ),
        in_specs=[
            pl.BlockSpec((B, H), lambda: (0, 0)),
            pl.BlockSpec((H, H), lambda: (0, 0)),
            pl.BlockSpec((1, H), lambda: (0, 0)),
            pl.BlockSpec((1, H), lambda: (0, 0)),
            pl.BlockSpec(memory_space=pltpu.SMEM),
        ],
        out_specs=(
            pl.BlockSpec((B, 1), lambda: (0, 0)),
            pl.BlockSpec((B, 1), lambda: (0, 0)),
            pl.BlockSpec((B, WP), lambda: (0, 0)),
        ),
    )(hidden, Wp_w, Wp_b.reshape(1, H), vp_w, vp_b.reshape(1, 1))

    table = encoder_outputs.reshape(S * B, H)
    enc_flat = _make_sc_gather()(idx2.reshape(B * WP), table)
    enc_local = enc_flat.reshape(B, WP, H)

    scaled_pad, ctx = pl.pallas_call(
        _attn_kernel,
        grid=(B // NB_B,),
        out_shape=(
            jax.ShapeDtypeStruct((B, 1, WP), jnp.float32),
            jax.ShapeDtypeStruct((B, 1, H), jnp.float32),
        ),
        in_specs=[
            pl.BlockSpec(memory_space=pltpu.SMEM),
            pl.BlockSpec(memory_space=pltpu.SMEM),
            pl.BlockSpec((NB_B, 1, H), lambda b: (b, 0, 0)),
            pl.BlockSpec((NB_B, WP, H), lambda b: (b, 0, 0)),
        ],
        out_specs=(
            pl.BlockSpec((NB_B, 1, WP), lambda b: (b, 0, 0)),
            pl.BlockSpec((NB_B, 1, H), lambda b: (b, 0, 0)),
        ),
        compiler_params=pltpu.CompilerParams(
            dimension_semantics=("parallel",)),
    )(p2, c2, hidden.reshape(B, 1, H), enc_local)

    return scaled_pad.reshape(B, WP)[:, :W], ctx.reshape(B, H)
